# Initial kernel scaffold; baseline (speedup 1.0000x reference)
import jax
import jax.numpy as jnp
from jax.experimental import pallas as pl


def kernel(gen_atom_table, gen_bond_table, gen_W1, gen_b1, gen_W2, gen_b2, gen_out_W, gen_out_b, disc_atom_table, disc_bond_table, disc_W1, disc_b1, disc_W2, disc_b2, disc_out_W, disc_out_b, node_feat, true_feat, edge_feat, edge_index, mask):
    raise NotImplementedError("write your pallas kernel here")


# speedup vs baseline: 1.3347x; 1.3347x over previous
import functools

import numpy as np
import jax
import jax.numpy as jnp
from jax import lax
from jax.experimental import pallas as pl
from jax.experimental.pallas import tpu as pltpu
from jax.experimental.pallas import tpu_sc as plsc

_ATOM_DIMS = [119, 4, 12, 12, 10, 6, 6, 2, 2]
_BOND_DIMS = [5, 6, 2]
_CS = np.concatenate([[0], np.cumsum(_ATOM_DIMS)]).astype(np.int32)
_GEN_OFF = np.concatenate([[0], np.cumsum([d + 1 for d in _ATOM_DIMS])[:-1]]).astype(np.int32)
_DISC_OFF = np.concatenate([[0], np.cumsum(_ATOM_DIMS)[:-1]]).astype(np.int32)
_BOND_OFF = np.concatenate([[0], np.cumsum(_BOND_DIMS)[:-1]]).astype(np.int32)

_N = 10000
_E = 320000
_N_PAD = 10240
_CK = 128
_E_PAD = _E + 4 * _CK

_F32 = jnp.float32
_I32 = jnp.int32

_OHB = np.zeros((8, 16), np.float32)
for _c in range(8):
    _OHB[_c, _BOND_OFF[0] + ((_c >> 2) & 1)] += 1.0
    _OHB[_c, _BOND_OFF[1] + ((_c >> 1) & 1)] += 1.0
    _OHB[_c, _BOND_OFF[2] + (_c & 1)] += 1.0


def _pad2(x, r, c):
    return jnp.pad(x, ((0, r - x.shape[0]), (0, c - x.shape[1])))


def _disc_prep_body(ohb_ref, bd_tbl_ref, dc0_ref, dc1_ref, db1_ref,
                    ctd0_ref, ctd1_ref):
    ed = jnp.dot(ohb_ref[...], bd_tbl_ref[...], preferred_element_type=_F32,
                 precision=lax.Precision.HIGHEST)
    ctd0_ref[...] = jnp.dot(ed, dc0_ref[...], preferred_element_type=_F32,
                            precision=lax.Precision.HIGHEST) + db1_ref[0:1, :]
    ctd1_ref[...] = jnp.dot(ed, dc1_ref[...], preferred_element_type=_F32,
                            precision=lax.Precision.HIGHEST) + db1_ref[1:2, :]


def _disc_prep(ohb, bd_tbl, dc0, dc1, db1):
    outs = (
        jax.ShapeDtypeStruct((8, 128), _F32),
        jax.ShapeDtypeStruct((8, 128), _F32),
    )
    return pl.pallas_call(_disc_prep_body, out_shape=outs)(
        ohb, bd_tbl, dc0, dc1, db1)


def _deg_stats_body(cnt_ref, stats_ref):
    deg = cnt_ref[:, 0:1]
    row = lax.broadcasted_iota(_I32, (_N_PAD, 1), 0)
    real = (row < _N).astype(_F32)
    ldg = jnp.log(deg + 1.0)
    avg = jnp.sum(ldg * real) / np.float32(_N)
    amp = ldg / avg
    att = avg / jnp.maximum(ldg, 1e-3)
    cntc = jnp.maximum(deg, 1.0)
    has = (deg > 0.0).astype(_F32)
    ones = jnp.ones((1, 128), _F32)
    stats_ref[...] = jnp.concatenate(
        [cntc * ones, has * ones, amp * ones, att * ones], axis=1)


def _deg_stats(cnt):
    return pl.pallas_call(
        _deg_stats_body,
        out_shape=jax.ShapeDtypeStruct((_N_PAD, 512), _F32),
    )(cnt)


def _pna_post_body(s_ref, q_ref, mx_ref, mn_ref, h_ref, st_ref, w2_ref, b2_ref,
                   wa_ref, wb_ref, hn_ref, an_ref, bn_ref, *, last):
    cntc = st_ref[:, 0:1]
    has = st_ref[:, 128:129]
    amp = st_ref[:, 256:257]
    att = st_ref[:, 384:385]
    s = s_ref[...]
    mean = s / cntc
    q = q_ref[...]
    std = jnp.sqrt(jnp.maximum(q / cntc - mean * mean, 0.0) + 1e-5)
    mx = mx_ref[...]
    mn = jnp.where(has > 0.0, mn_ref[...], 0.0)
    agg = jnp.concatenate([mean, mx, mn, std], axis=1)
    h = h_ref[...]
    x = jnp.concatenate([h, agg, agg * amp, agg * att], axis=1)
    hn = jnp.maximum(jnp.dot(x, w2_ref[...], preferred_element_type=_F32,
                             precision=lax.Precision.HIGHEST)
                     + b2_ref[0:1, :], 0.0) + h
    hn_ref[...] = hn
    if not last:
        an_ref[...] = jnp.dot(hn, wa_ref[...], preferred_element_type=_F32,
                              precision=lax.Precision.HIGHEST)
        bn_ref[...] = jnp.dot(hn, wb_ref[...], preferred_element_type=_F32,
                              precision=lax.Precision.HIGHEST)


@functools.lru_cache(maxsize=None)
def _pna_post_fn(hd, bn, last):
    grid = _N_PAD // bn
    nblk = lambda i: (i, 0)
    full = lambda i: (0, 0)
    in_specs = [
        pl.BlockSpec((bn, hd), nblk),
        pl.BlockSpec((bn, hd), nblk),
        pl.BlockSpec((bn, hd), nblk),
        pl.BlockSpec((bn, hd), nblk),
        pl.BlockSpec((bn, hd), nblk),
        pl.BlockSpec((bn, 512), nblk),
        pl.BlockSpec((13 * hd, hd), full),
        pl.BlockSpec((8, hd), full),
        pl.BlockSpec((hd, hd), full),
        pl.BlockSpec((hd, hd), full),
    ]
    out_specs = (
        pl.BlockSpec((bn, hd), nblk),
        pl.BlockSpec((bn, hd), nblk),
        pl.BlockSpec((bn, hd), nblk),
    )
    outs = (
        jax.ShapeDtypeStruct((_N_PAD, hd), _F32),
        jax.ShapeDtypeStruct((_N_PAD, hd), _F32),
        jax.ShapeDtypeStruct((_N_PAD, hd), _F32),
    )
    return pl.pallas_call(
        functools.partial(_pna_post_body, last=last),
        grid=(grid,),
        in_specs=in_specs,
        out_specs=out_specs,
        out_shape=outs,
    )


def _readout_body(h_ref, ow_ref, ob_ref, tf_ref, nfn_ref, mf_ref, dt_ref,
                  wa_ref, wb_ref, h2_ref, a0_ref, b0_ref, loss_ref):
    i = pl.program_id(0)
    nb = pl.num_programs(0)
    recon = jnp.dot(h_ref[...], ow_ref[...], preferred_element_type=_F32,
                    precision=lax.Precision.HIGHEST) + ob_ref[0:1, :]
    bnr = recon.shape[0]
    col = lax.broadcasted_iota(_I32, (bnr, 256), 1)
    maskf = mf_ref[:, 0:1]
    nll = jnp.zeros((bnr, 1), _F32)
    oh = jnp.zeros((bnr, 256), _F32)
    for g in range(9):
        lo = int(_CS[g])
        hi = int(_CS[g + 1])
        gm = (col >= lo) & (col < hi)
        neg = jnp.where(gm, recon, -1e30)
        mxv = jnp.max(neg, axis=1, keepdims=True)
        ex = jnp.where(gm, jnp.exp(recon - mxv), 0.0)
        lse = mxv + jnp.log(jnp.sum(ex, axis=1, keepdims=True))
        t = tf_ref[:, g:g + 1]
        tl = jnp.sum(jnp.where(col == lo + t, recon, 0.0), axis=1, keepdims=True)
        nll = nll + (lse - tl)
        ridx = int(_DISC_OFF[g]) + nfn_ref[:, g:g + 1]
        oh = oh + jnp.where(col == ridx, 1.0, 0.0)
    valid = 1.0 - maskf
    bn_num = jnp.sum(valid * nll)
    bn_v = jnp.sum(valid)
    h2 = jnp.dot(oh, dt_ref[...], preferred_element_type=_F32,
                 precision=lax.Precision.HIGHEST)
    h2_ref[...] = h2
    a0_ref[...] = jnp.dot(h2, wa_ref[...], preferred_element_type=_F32,
                          precision=lax.Precision.HIGHEST)
    b0_ref[...] = jnp.dot(h2, wb_ref[...], preferred_element_type=_F32,
                          precision=lax.Precision.HIGHEST)

    r8 = lax.broadcasted_iota(_I32, (8, 128), 0)
    c8 = lax.broadcasted_iota(_I32, (8, 128), 1)
    m00 = (r8 == 0) & (c8 == 0)
    m01 = (r8 == 0) & (c8 == 1)

    @pl.when(i == 0)
    def _():
        loss_ref[...] = jnp.zeros_like(loss_ref)

    loss_ref[...] = (loss_ref[...] + jnp.where(m00, bn_num, 0.0)
                     + jnp.where(m01, bn_v, 0.0))

    @pl.when(i == nb - 1)
    def _():
        acc = loss_ref[...]
        num = jnp.sum(jnp.where(m00, acc, 0.0))
        v = jnp.sum(jnp.where(m01, acc, 0.0))
        loss_ref[...] = jnp.where(m00, num / jnp.maximum(v, 1.0), acc)


@functools.lru_cache(maxsize=None)
def _readout_fn(hg, hd, bn):
    grid = _N_PAD // bn
    nblk = lambda i: (i, 0)
    full = lambda i: (0, 0)
    in_specs = [
        pl.BlockSpec((bn, hg), nblk),
        pl.BlockSpec((hg, 256), full),
        pl.BlockSpec((8, 256), full),
        pl.BlockSpec((bn, 16), nblk),
        pl.BlockSpec((bn, 16), nblk),
        pl.BlockSpec((bn, 128), nblk),
        pl.BlockSpec((256, hd), full),
        pl.BlockSpec((hd, hd), full),
        pl.BlockSpec((hd, hd), full),
    ]
    out_specs = (
        pl.BlockSpec((bn, hd), nblk),
        pl.BlockSpec((bn, hd), nblk),
        pl.BlockSpec((bn, hd), nblk),
        pl.BlockSpec((8, 128), full),
    )
    outs = (
        jax.ShapeDtypeStruct((_N_PAD, hd), _F32),
        jax.ShapeDtypeStruct((_N_PAD, hd), _F32),
        jax.ShapeDtypeStruct((_N_PAD, hd), _F32),
        jax.ShapeDtypeStruct((8, 128), _F32),
    )
    return pl.pallas_call(
        _readout_body,
        grid=(grid,),
        in_specs=in_specs,
        out_specs=out_specs,
        out_shape=outs,
    )


def _disc_readout_body(h_ref, w_ref, b_ref, out_ref):
    out_ref[...] = jnp.dot(h_ref[...], w_ref[...], preferred_element_type=_F32,
                           precision=lax.Precision.HIGHEST) + b_ref[0:1, :]


def _disc_readout(h, w, b):
    return pl.pallas_call(
        _disc_readout_body,
        out_shape=jax.ShapeDtypeStruct((_N_PAD, 128), _F32),
    )(h, w, b)


@functools.lru_cache(maxsize=None)
def _sc_edge_fn(h, vw):
    npw = _N_PAD // vw
    nw = 32
    vw_per_w = vw // nw
    fg = h // 16
    mesh = plsc.VectorSubcoreMesh(core_axis_name="c", subcore_axis_name="s",
                                  num_cores=2, num_subcores=16)
    outs = [jax.ShapeDtypeStruct((_N_PAD, h), _F32) for _ in range(4)]
    scratch = [
        pltpu.VMEM((npw, h), _F32),
        pltpu.VMEM((npw, h), _F32),
        pltpu.VMEM((npw, h), _F32),
        pltpu.VMEM((npw, h), _F32),
        pltpu.VMEM((npw, h), _F32),
        pltpu.VMEM((_CK, h), _F32),
        pltpu.VMEM((_CK,), _I32),
        pltpu.VMEM((_CK,), _I32),
        pltpu.VMEM((_CK,), _I32),
        pltpu.VMEM((8, h), _F32),
        pltpu.VMEM((160,), _I32),
        pltpu.SemaphoreType.DMA,
    ]

    def body(a_hbm, b_hbm, ctab_hbm, src_hbm, dst_hbm, code_hbm, wb_hbm,
             s_out, q_out, mx_out, mn_out,
             s_v, q_v, mx_v, mn_v, b_v, a_v, src_v, dst_v, code_v, ct_v, wb_v,
             sem):
        wid = lax.axis_index("s") * 2 + lax.axis_index("c")
        pltpu.sync_copy(wb_hbm, wb_v)
        pltpu.sync_copy(ctab_hbm, ct_v)
        zero16 = jnp.zeros((16,), _F32)
        big16 = jnp.full((16,), 1e30, _F32)
        for sub in range(vw_per_w):
            v = wid * vw_per_w + sub
            base = v * npw

            def initrow(r, _):
                for f in range(fg):
                    sl = pl.ds(f * 16, 16)
                    s_v[r, sl] = zero16
                    q_v[r, sl] = zero16
                    mx_v[r, sl] = zero16
                    mn_v[r, sl] = big16
                return 0

            lax.fori_loop(0, npw, initrow, 0)
            pltpu.sync_copy(b_hbm.at[pl.ds(base, npw)], b_v)
            wbc = wb_v[pl.ds(v, 16)]
            e0 = wbc[0]
            e1 = wbc[1]
            ea = (e0 // 8) * 8
            nch = (e1 - ea + (_CK - 1)) // _CK

            def chunk(kc, _):
                cb = ea + kc * _CK
                pltpu.sync_copy(src_hbm.at[pl.ds(cb, _CK)], src_v)
                pltpu.sync_copy(dst_hbm.at[pl.ds(cb, _CK)], dst_v)
                pltpu.sync_copy(code_hbm.at[pl.ds(cb, _CK)], code_v)
                pltpu.async_copy(a_hbm.at[src_v], a_v, sem).wait()

                def group(g, _):
                    dv = dst_v[pl.ds(g * 16, 16)] - base
                    cv = code_v[pl.ds(g * 16, 16)]
                    for j in range(16):
                        e = cb + g * 16 + j

                        @pl.when((e >= e0) & (e < e1))
                        def _():
                            ld = dv[j]
                            cd = cv[j]
                            for f in range(fg):
                                sl = pl.ds(f * 16, 16)
                                m = jnp.maximum(
                                    a_v[g * 16 + j, sl] + b_v[ld, sl]
                                    + ct_v[cd, sl], 0.0)
                                s_v[ld, sl] = s_v[ld, sl] + m
                                q_v[ld, sl] = q_v[ld, sl] + m * m
                                mx_v[ld, sl] = jnp.maximum(mx_v[ld, sl], m)
                                mn_v[ld, sl] = jnp.minimum(mn_v[ld, sl], m)

                    return 0

                lax.fori_loop(0, _CK // 16, group, 0)
                return 0

            lax.fori_loop(0, nch, chunk, 0)
            pltpu.sync_copy(s_v, s_out.at[pl.ds(base, npw)])
            pltpu.sync_copy(q_v, q_out.at[pl.ds(base, npw)])
            pltpu.sync_copy(mx_v, mx_out.at[pl.ds(base, npw)])
            pltpu.sync_copy(mn_v, mn_out.at[pl.ds(base, npw)])

    return pl.kernel(body, out_type=tuple(outs), mesh=mesh,
                     scratch_types=tuple(scratch))


def kernel(gen_atom_table, gen_bond_table, gen_W1, gen_b1, gen_W2, gen_b2,
           gen_out_W, gen_out_b, disc_atom_table, disc_bond_table, disc_W1,
           disc_b1, disc_W2, disc_b2, disc_out_W, disc_out_b, node_feat,
           true_feat, edge_feat, edge_index, mask):
    hg = gen_atom_table.shape[1]
    hd = disc_atom_table.shape[1]
    n = node_feat.shape[0]
    src0, dst0 = edge_index[0], edge_index[1]

    deg = jnp.zeros((n,), jnp.float32).at[dst0].add(1.0)
    log_deg = jnp.log(deg + 1.0)
    avg_d = jnp.mean(log_deg)
    goff = jnp.asarray(_GEN_OFF)
    boff = jnp.asarray(_BOND_OFF)
    h = jnp.sum(gen_atom_table[node_feat + goff[None, :]], axis=1)
    e = jnp.sum(gen_bond_table[edge_feat + boff[None, :]], axis=1)
    for l in range(gen_W1.shape[0]):
        m = jax.nn.relu(jnp.concatenate([h[src0], h[dst0], e], axis=1)
                        @ gen_W1[l] + gen_b1[l])
        cnt = jnp.maximum(deg, 1.0)[:, None]
        s = jax.ops.segment_sum(m, dst0, num_segments=n)
        mean = s / cnt
        sq = jax.ops.segment_sum(m * m, dst0, num_segments=n)
        std = jnp.sqrt(jnp.maximum(sq / cnt - mean * mean, 0.0) + 1e-5)
        has = (deg > 0)[:, None]
        mx = jnp.where(has, jax.ops.segment_max(m, dst0, num_segments=n), 0.0)
        mn = jnp.where(has, -jax.ops.segment_max(-m, dst0, num_segments=n), 0.0)
        agg = jnp.concatenate([mean, mx, mn, std], axis=1)
        amp = (log_deg / avg_d)[:, None]
        att = (avg_d / jnp.maximum(log_deg, 1e-3))[:, None]
        out = jnp.concatenate([agg, agg * amp, agg * att], axis=1)
        h = jax.nn.relu(jnp.concatenate([h, out], axis=1) @ gen_W2[l]
                        + gen_b2[l]) + h
    recon = h @ gen_out_W + gen_out_b
    fakes = []
    for i in range(9):
        logits = recon[:, int(_CS[i]):int(_CS[i + 1])]
        probs = jax.nn.softmax(jax.lax.stop_gradient(logits), axis=1)
        fakes.append(jnp.argmax(probs, axis=1))
    fake = jnp.stack(fakes, axis=1)
    new_feat = jnp.where(mask, true_feat, fake)

    src = src0.astype(_I32)
    dst = dst0.astype(_I32)
    code = (edge_feat[:, 0] * 4 + edge_feat[:, 1] * 2 + edge_feat[:, 2]).astype(_I32)
    dst_s, src_s, code_s = lax.sort((dst, src, code), num_keys=1, is_stable=True)
    padn = _E_PAD - _E
    dst_s = jnp.concatenate([dst_s, jnp.full((padn,), _N_PAD, _I32)])
    src_s = jnp.concatenate([src_s, (jnp.arange(padn, dtype=_I32) % _N)])
    code_s = jnp.concatenate([code_s, jnp.zeros((padn,), _I32)])

    npw = _N_PAD // 128
    bounds = jnp.arange(0, _N_PAD + 1, npw, dtype=_I32)
    wb = jnp.zeros((160,), _I32).at[:129].set(
        jnp.searchsorted(dst_s[:_E], bounds).astype(_I32))

    ohb = jnp.asarray(_OHB)
    bd_tbl = _pad2(disc_bond_table, 16, hd)
    d_w1a = [disc_W1[l, 0:hd, :] for l in range(2)]
    d_w1b = [disc_W1[l, hd:2 * hd, :] for l in range(2)]
    d_w1c = [disc_W1[l, 2 * hd:3 * hd, :] for l in range(2)]
    db1 = _pad2(disc_b1, 8, hd)
    ctd0, ctd1 = _disc_prep(ohb, bd_tbl, d_w1c[0], d_w1c[1], db1)

    stats = _deg_stats(
        jnp.pad(jnp.broadcast_to(deg[:, None], (n, 16)),
                ((0, _N_PAD - n), (0, 0))))

    hpad = _pad2(h, _N_PAD, hg)
    ow = _pad2(gen_out_W, hg, 256)
    ob = _pad2(gen_out_b[None, :], 8, 256)
    tfp = _pad2(true_feat.astype(_I32), _N_PAD, 16)
    nfn = _pad2(new_feat.astype(_I32), _N_PAD, 16)
    mf = jnp.broadcast_to(
        jnp.pad(mask.astype(_F32), ((0, _N_PAD - n), (0, 0)),
                constant_values=1.0), (_N_PAD, 128))
    dt = _pad2(disc_atom_table, 256, hd)
    hD, aD, bD, loss = _readout_fn(hg, hd, 2048)(
        hpad, ow, ob, tfp, nfn, mf, dt, d_w1a[0], d_w1b[0])

    d_b2 = [_pad2(disc_b2[l:l + 1], 8, hd) for l in range(2)]

    s, q, mx, mn = _sc_edge_fn(hd, 128)(
        aD, bD, ctd0, src_s, dst_s, code_s, wb)
    hD1, aD1, bD1 = _pna_post_fn(hd, 1024, False)(
        s, q, mx, mn, hD, stats, disc_W2[0], d_b2[0], d_w1a[1], d_w1b[1])

    s, q, mx, mn = _sc_edge_fn(hd, 128)(
        aD1, bD1, ctd1, src_s, dst_s, code_s, wb)
    hD2, _, _ = _pna_post_fn(hd, 1024, True)(
        s, q, mx, mn, hD1, stats, disc_W2[1], d_b2[1], d_w1a[1], d_w1b[1])

    pred = _disc_readout(hD2, _pad2(disc_out_W, hd, 128),
                         _pad2(disc_out_b[None, :], 8, 128))

    gen_loss = loss[0, 0]
    predictions = pred[:_N, 0:1]
    return gen_loss, predictions
